# Initial kernel scaffold; baseline (speedup 1.0000x reference)
import jax
import jax.numpy as jnp
from jax.experimental import pallas as pl


def kernel(x, edge_attr, edge_index, batch, W_enc, b_enc, W_self0, W_nbr0, b0, W_self1, W_nbr1, b1, W_self2, W_nbr2, b2, W_dec, b_dec):
    raise NotImplementedError("write your pallas kernel here")


# speedup vs baseline: 5.0968x; 5.0968x over previous
import functools

import jax
import jax.numpy as jnp
from jax import lax
from jax.experimental import pallas as pl
from jax.experimental.pallas import tpu as pltpu
from jax.experimental.pallas import tpu_sc as plsc

N = 10000
D = 128
H = 128
E = 320000
G = 64
OUT = 10

NC = 2
NS = 16
NW = NC * NS
EDGES_PER_W = E // NW
K = 80
NCHUNK = EDGES_PER_W // K
ROWS_PER_SUB = 632
NP = ROWS_PER_SUB * NS


def _make_agg():
    mesh = plsc.VectorSubcoreMesh(core_axis_name="c", subcore_axis_name="s")

    @functools.partial(
        pl.kernel,
        out_type=jax.ShapeDtypeStruct((NC * NP, H), jnp.float32),
        mesh=mesh,
        scratch_types=[
            pltpu.VMEM_SHARED((NP, H), jnp.float32),
            pltpu.VMEM((K,), jnp.int32),
            pltpu.VMEM((K,), jnp.int32),
            pltpu.VMEM((K, H), jnp.float32),
            pltpu.SemaphoreType.DMA,
        ],
    )
    def agg(m_hbm, src_hbm, dst_hbm, zero_hbm, out_hbm,
            acc, src_v, dst_v, rows_v, sem):
        c = lax.axis_index("c")
        s = lax.axis_index("s")
        wid = c * NS + s
        r0 = s * ROWS_PER_SUB
        pltpu.sync_copy(zero_hbm.at[pl.ds(r0, ROWS_PER_SUB)],
                        acc.at[pl.ds(r0, ROWS_PER_SUB)])
        plsc.subcore_barrier()
        base = wid * EDGES_PER_W

        def body(j, carry):
            off = base + j * K
            pltpu.sync_copy(src_hbm.at[pl.ds(off, K)], src_v)
            pltpu.sync_copy(dst_hbm.at[pl.ds(off, K)], dst_v)
            pltpu.async_copy(m_hbm.at[src_v], rows_v, sem).wait()
            pltpu.sync_copy(rows_v, acc.at[dst_v], add=True)
            return carry

        lax.fori_loop(0, NCHUNK, body, 0)
        plsc.subcore_barrier()
        pltpu.sync_copy(acc.at[pl.ds(r0, ROWS_PER_SUB)],
                        out_hbm.at[pl.ds(c * NP + r0, ROWS_PER_SUB)])

    return agg


_agg = _make_agg()


def _enc_body(x_ref, we_ref, be_ref, wn_ref, h_ref, m_ref):
    h = jnp.dot(x_ref[...], we_ref[...],
                preferred_element_type=jnp.float32) + be_ref[...]
    h_ref[...] = h
    m_ref[...] = jnp.dot(h, wn_ref[...], preferred_element_type=jnp.float32)


_enc = pl.pallas_call(
    _enc_body,
    out_shape=(jax.ShapeDtypeStruct((N, H), jnp.float32),
               jax.ShapeDtypeStruct((N, H), jnp.float32)),
)


def _mid_body(h_ref, a_ref, ws_ref, b_ref, wnn_ref, hn_ref, mn_ref):
    t = jnp.dot(h_ref[...], ws_ref[...], preferred_element_type=jnp.float32)
    t = t + a_ref[0:N, :] + a_ref[NP:NP + N, :] + b_ref[...]
    t = jnp.maximum(t, 0.0)
    hn_ref[...] = t
    mn_ref[...] = jnp.dot(t, wnn_ref[...], preferred_element_type=jnp.float32)


_mid = pl.pallas_call(
    _mid_body,
    out_shape=(jax.ShapeDtypeStruct((N, H), jnp.float32),
               jax.ShapeDtypeStruct((N, H), jnp.float32)),
)


def _fin_body(h_ref, a_ref, ws_ref, b_ref, batch_ref, wd_ref, bd_ref, out_ref):
    h3 = jnp.dot(h_ref[...], ws_ref[...], preferred_element_type=jnp.float32)
    h3 = h3 + a_ref[0:N, :] + a_ref[NP:NP + N, :] + b_ref[...]
    seg = lax.broadcasted_iota(jnp.int32, (G, N), 0)
    onehot = (seg == batch_ref[...]).astype(jnp.float32)
    g = jnp.dot(onehot, h3, preferred_element_type=jnp.float32)
    out_ref[...] = jnp.dot(g, wd_ref[...],
                           preferred_element_type=jnp.float32) + bd_ref[...]


_fin = pl.pallas_call(
    _fin_body,
    out_shape=jax.ShapeDtypeStruct((G, OUT), jnp.float32),
)


def kernel(x, edge_attr, edge_index, batch, W_enc, b_enc,
           W_self0, W_nbr0, b0, W_self1, W_nbr1, b1,
           W_self2, W_nbr2, b2, W_dec, b_dec):
    src = edge_index[0]
    dst = edge_index[1]
    zeros = jnp.zeros((NP, H), jnp.float32)
    batch2 = batch.reshape(1, N)

    h0, m0 = _enc(x, W_enc, b_enc.reshape(1, H), W_nbr0)
    a0 = _agg(m0, src, dst, zeros)
    h1, m1 = _mid(h0, a0, W_self0, b0.reshape(1, H), W_nbr1)
    a1 = _agg(m1, src, dst, zeros)
    h2, m2 = _mid(h1, a1, W_self1, b1.reshape(1, H), W_nbr2)
    a2 = _agg(m2, src, dst, zeros)
    out = _fin(h2, a2, W_self2, b2.reshape(1, H), batch2, W_dec,
               b_dec.reshape(1, OUT))
    return out
